# Initial kernel scaffold; baseline (speedup 1.0000x reference)
import jax
import jax.numpy as jnp
from jax.experimental import pallas as pl


def kernel(ids, feats, adjs, edge_emb, W_self_0, W_neigh_0, W_self_1, W_neigh_1, fc_w, fc_b):
    raise NotImplementedError("write your pallas kernel here")


# speedup vs baseline: 7.2749x; 7.2749x over previous
import functools

import jax
import jax.numpy as jnp
from jax import lax
from jax.experimental import pallas as pl
from jax.experimental.pallas import tpu as pltpu
from jax.experimental.pallas import tpu_sc as plsc

B = 8192
N_NODES = 50000
DEG = 32
D = 128
D_E = 16
N_CLASSES = 16
S = 25
OFF = DEG - S

NC, NS, L = 2, 16, 16
NW = NC * NS
BPW = B // NW
CH = 4
HALF = BPW // 2
NCH = HALF // CH


def _sc_aggregate(ids, feats, adjs2, et_exp):
    mesh = plsc.VectorSubcoreMesh(
        core_axis_name="c", subcore_axis_name="s",
        num_cores=NC, num_subcores=NS)

    @functools.partial(
        pl.kernel,
        out_type=[
            jax.ShapeDtypeStruct((B, D), jnp.float32),
            jax.ShapeDtypeStruct((B, D), jnp.float32),
            jax.ShapeDtypeStruct((B, D), jnp.float32),
            jax.ShapeDtypeStruct((B * D_E,), jnp.float32),
            jax.ShapeDtypeStruct((B * D_E,), jnp.float32),
        ],
        mesh=mesh,
        compiler_params=pltpu.CompilerParams(needs_layout_passes=False),
        scratch_types=[
            pltpu.VMEM((BPW,), jnp.int32),
            pltpu.VMEM((CH * DEG, D), jnp.int32),
            pltpu.VMEM((CH * DEG * D,), jnp.int32),
            pltpu.VMEM((BPW * DEG,), jnp.int32),
            pltpu.VMEM((CH * DEG, D), jnp.float32),
            pltpu.VMEM((CH * DEG, D), jnp.float32),
            pltpu.VMEM((HALF, D), jnp.float32),
            pltpu.VMEM((HALF, D), jnp.float32),
            pltpu.VMEM((HALF * D_E,), jnp.float32),
            pltpu.VMEM((HALF * D_E,), jnp.float32),
            pltpu.SemaphoreType.DMA,
            pltpu.SemaphoreType.DMA,
        ],
    )
    def sc_kernel(ids_hbm, feats_hbm, adjs2_hbm, et_hbm,
                  f0_out, m0f_out, m1f_out, m0e_out, m1e_out,
                  ids_v, abuf, abuf_f, adjf_v, fbuf, ebuf,
                  m0f_v, m1f_v, m0e_v, m1e_v, sem_f, sem_e):
        wid = lax.axis_index("s") * NC + lax.axis_index("c")
        base = wid * BPW
        iota = lax.iota(jnp.int32, 16)

        pltpu.sync_copy(ids_hbm.at[pl.ds(base, BPW)], ids_v)

        for h in range(BPW // 128):
            pltpu.async_copy(
                feats_hbm.at[ids_v.at[pl.ds(h * 128, 128)]], fbuf, sem_f
            ).wait()
            pltpu.sync_copy(fbuf, f0_out.at[pl.ds(base + h * 128, 128)])

        for h in range(BPW // 128):
            def sup_body(g, carry):
                v = ids_v[pl.ds(h * 128 + g * 16, 16)]
                adjf_v[pl.ds(h * 128 * DEG + g * 16, 16)] = (
                    lax.shift_right_logical(v, 2))
                return carry
            lax.fori_loop(0, 8, sup_body, 0)
            pltpu.async_copy(
                adjs2_hbm.at[adjf_v.at[pl.ds(h * 128 * DEG, 128)]], abuf, sem_f
            ).wait()

            def flat_body(g, carry):
                for rr in range(2):
                    r = g * 2 + rr
                    for k in range(D // L):
                        abuf_f[pl.ds(r * D + k * L, L)] = abuf[r, pl.ds(k * L, L)]
                return carry
            lax.fori_loop(0, 64, flat_body, 0)

            def ext_body(g, carry):
                idv = ids_v[pl.ds(h * 128 + g * 16, 16)]
                srcbase = (g * 16 + iota) * D + (idv & 3) * DEG
                dst_base = (h * 128 + g * 16 + iota) * DEG
                for t in range(DEG):
                    vals = plsc.load_gather(abuf_f, [srcbase + t])
                    plsc.store_scatter(adjf_v, [dst_base + t], vals)
                return carry
            lax.fori_loop(0, 8, ext_body, 0)

        for half in range(2):
            def chunk_body(j, carry):
                l0 = j * CH
                e0 = (half * HALF) * DEG + l0 * DEG
                idx = adjf_v.at[pl.ds(e0, CH * DEG)]
                cpf = pltpu.async_copy(feats_hbm.at[idx], fbuf, sem_f)
                cpe = pltpu.async_copy(et_hbm.at[idx], ebuf, sem_e)
                cpf.wait()
                cpe.wait()
                for s in range(CH):
                    r0 = s * DEG
                    orow = l0 + s
                    for k in range(D // L):
                        sl = pl.ds(k * L, L)
                        accA = fbuf[r0 + 0, sl]
                        for t in range(1, OFF):
                            accA = accA + fbuf[r0 + t, sl]
                        accB = fbuf[r0 + OFF, sl]
                        for t in range(OFF + 1, S):
                            accB = accB + fbuf[r0 + t, sl]
                        accC = fbuf[r0 + S, sl]
                        for t in range(S + 1, DEG):
                            accC = accC + fbuf[r0 + t, sl]
                        m0f_v[orow, sl] = accA + accB
                        m1f_v[orow, sl] = accB + accC
                    esl = pl.ds(0, L)
                    eA = ebuf[r0 + 0, esl]
                    for t in range(1, OFF):
                        eA = eA + ebuf[r0 + t, esl]
                    eB = ebuf[r0 + OFF, esl]
                    for t in range(OFF + 1, S):
                        eB = eB + ebuf[r0 + t, esl]
                    eC = ebuf[r0 + S, esl]
                    for t in range(S + 1, DEG):
                        eC = eC + ebuf[r0 + t, esl]
                    m0e_v[pl.ds(orow * D_E, D_E)] = eA + eB
                    m1e_v[pl.ds(orow * D_E, D_E)] = eB + eC
                return carry
            lax.fori_loop(0, NCH, chunk_body, 0)
            row0 = base + half * HALF
            pltpu.sync_copy(m0f_v, m0f_out.at[pl.ds(row0, HALF)])
            pltpu.sync_copy(m1f_v, m1f_out.at[pl.ds(row0, HALF)])
            pltpu.sync_copy(m0e_v, m0e_out.at[pl.ds(row0 * D_E, HALF * D_E)])
            pltpu.sync_copy(m1e_v, m1e_out.at[pl.ds(row0 * D_E, HALF * D_E)])

    return sc_kernel(ids, feats, adjs2, et_exp)


def _dense_tail(f0, m0f, m1f, m0e, m1e,
                W_self_0, W_neigh_0, W_self_1, W_neigh_1, fc_w, fc_b2):
    BT = 1024
    H = D // 2

    def body(f0_r, m0f_r, m1f_r, m0e_r, m1e_r,
             ws0_r, wn0_r, ws1_r, wn1_r, fcw_r, fcb_r, out_r):
        inv_s = jnp.float32(1.0 / S)
        a0 = jnp.maximum(
            jnp.dot(f0_r[...], ws0_r[...], preferred_element_type=jnp.float32), 0.0)
        a1 = jnp.maximum(
            jnp.dot(f0_r[...], ws1_r[...], preferred_element_type=jnp.float32), 0.0)
        b0 = jnp.maximum(
            (jnp.dot(m0f_r[...], wn0_r[0:D, :], preferred_element_type=jnp.float32)
             + jnp.dot(m0e_r[...], wn0_r[D:D + D_E, :],
                       preferred_element_type=jnp.float32)) * inv_s, 0.0)
        b1 = jnp.maximum(
            (jnp.dot(m1f_r[...], wn1_r[0:D, :], preferred_element_type=jnp.float32)
             + jnp.dot(m1e_r[...], wn1_r[D:D + D_E, :],
                       preferred_element_type=jnp.float32)) * inv_s, 0.0)
        ha = (a0 + a1) * 0.5
        hb = (b0 + b1) * 0.5
        out_r[...] = jnp.maximum(
            jnp.dot(ha, fcw_r[0:H, :], preferred_element_type=jnp.float32)
            + jnp.dot(hb, fcw_r[H:D, :], preferred_element_type=jnp.float32)
            + fcb_r[...], 0.0)

    row_spec_d = pl.BlockSpec((BT, D), lambda i: (i, 0))
    row_spec_e = pl.BlockSpec((BT, D_E), lambda i: (i, 0))
    full = lambda shape: pl.BlockSpec(shape, lambda i: (0, 0))
    return pl.pallas_call(
        body,
        grid=(B // BT,),
        in_specs=[
            row_spec_d, row_spec_d, row_spec_d, row_spec_e, row_spec_e,
            full((D, H)), full((D + D_E, H)),
            full((D, H)), full((D + D_E, H)),
            full((D, N_CLASSES)), full((1, N_CLASSES)),
        ],
        out_specs=pl.BlockSpec((BT, N_CLASSES), lambda i: (i, 0)),
        out_shape=jax.ShapeDtypeStruct((B, N_CLASSES), jnp.float32),
    )(f0, m0f, m1f, m0e, m1e,
      W_self_0, W_neigh_0, W_self_1, W_neigh_1, fc_w, fc_b2)


def kernel(ids, feats, adjs, edge_emb,
           W_self_0, W_neigh_0, W_self_1, W_neigh_1, fc_w, fc_b):
    ids = ids.astype(jnp.int32)
    adjs2 = adjs.astype(jnp.int32).reshape(N_NODES * DEG // 128, 128)
    et_exp = jnp.tile(edge_emb[:N_NODES], (1, D // D_E))
    f0, m0f, m1f, m0e, m1e = _sc_aggregate(ids, feats, adjs2, et_exp)
    return _dense_tail(f0, m0f, m1f,
                       m0e.reshape(B, D_E), m1e.reshape(B, D_E),
                       W_self_0, W_neigh_0, W_self_1, W_neigh_1,
                       fc_w, fc_b.reshape(1, N_CLASSES))


# speedup vs baseline: 9.4646x; 1.3010x over previous
import functools

import jax
import jax.numpy as jnp
from jax import lax
from jax.experimental import pallas as pl
from jax.experimental.pallas import tpu as pltpu
from jax.experimental.pallas import tpu_sc as plsc

B = 8192
N_NODES = 50000
DEG = 32
D = 128
D_E = 16
N_CLASSES = 16
S = 25
OFF = DEG - S

NC, NS, L = 2, 16, 16
NW = NC * NS
BPW = B // NW
CH = 2
HALF = BPW // 2
NCH = HALF // CH
PQ = 64


def _sc_aggregate(ids, feats, adjs2, et_exp):
    mesh = plsc.VectorSubcoreMesh(
        core_axis_name="c", subcore_axis_name="s",
        num_cores=NC, num_subcores=NS)

    @functools.partial(
        pl.kernel,
        out_type=[
            jax.ShapeDtypeStruct((B, D), jnp.float32),
            jax.ShapeDtypeStruct((B, D), jnp.float32),
            jax.ShapeDtypeStruct((B, D), jnp.float32),
            jax.ShapeDtypeStruct((B * D_E,), jnp.float32),
            jax.ShapeDtypeStruct((B * D_E,), jnp.float32),
        ],
        mesh=mesh,
        compiler_params=pltpu.CompilerParams(needs_layout_passes=False),
        scratch_types=[
            pltpu.VMEM((BPW,), jnp.int32),
            pltpu.VMEM((PQ, D), jnp.int32),
            pltpu.VMEM((PQ * D,), jnp.int32),
            pltpu.VMEM((BPW * DEG,), jnp.int32),
            pltpu.VMEM((CH * DEG, D), jnp.float32),
            pltpu.VMEM((CH * DEG, D), jnp.float32),
            pltpu.VMEM((CH * DEG, D), jnp.float32),
            pltpu.VMEM((CH * DEG, D), jnp.float32),
            pltpu.VMEM((HALF, D), jnp.float32),
            pltpu.VMEM((HALF, D), jnp.float32),
            pltpu.VMEM((HALF * D_E,), jnp.float32),
            pltpu.VMEM((HALF * D_E,), jnp.float32),
            pltpu.SemaphoreType.DMA,
            pltpu.SemaphoreType.DMA,
            pltpu.SemaphoreType.DMA,
            pltpu.SemaphoreType.DMA,
        ],
    )
    def sc_kernel(ids_hbm, feats_hbm, adjs2_hbm, et_hbm,
                  f0_out, m0f_out, m1f_out, m0e_out, m1e_out,
                  ids_v, abuf, abuf_f, adjf_v,
                  fbufA, fbufB, ebufA, ebufB,
                  m0f_v, m1f_v, m0e_v, m1e_v,
                  semAf, semAe, semBf, semBe):
        wid = lax.axis_index("s") * NC + lax.axis_index("c")
        base = wid * BPW
        iota = lax.iota(jnp.int32, 16)

        pltpu.sync_copy(ids_hbm.at[pl.ds(base, BPW)], ids_v)

        for h in range(BPW // PQ):
            pltpu.async_copy(
                feats_hbm.at[ids_v.at[pl.ds(h * PQ, PQ)]], fbufA, semAf
            ).wait()
            pltpu.sync_copy(fbufA, f0_out.at[pl.ds(base + h * PQ, PQ)])

        for h in range(BPW // PQ):
            def sup_body(g, carry):
                v = ids_v[pl.ds(h * PQ + g * 16, 16)]
                adjf_v[pl.ds(h * PQ * DEG + g * 16, 16)] = (
                    lax.shift_right_logical(v, 2))
                return carry
            lax.fori_loop(0, PQ // 16, sup_body, 0)
            pltpu.async_copy(
                adjs2_hbm.at[adjf_v.at[pl.ds(h * PQ * DEG, PQ)]], abuf, semAf
            ).wait()

            def flat_body(g, carry):
                for rr in range(2):
                    r = g * 2 + rr
                    for k in range(D // L):
                        abuf_f[pl.ds(r * D + k * L, L)] = abuf[r, pl.ds(k * L, L)]
                return carry
            lax.fori_loop(0, PQ // 2, flat_body, 0)

            def ext_body(g, carry):
                idv = ids_v[pl.ds(h * PQ + g * 16, 16)]
                srcbase = (g * 16 + iota) * D + (idv & 3) * DEG
                dst_base = (h * PQ + g * 16 + iota) * DEG
                for t in range(DEG):
                    vals = plsc.load_gather(abuf_f, [srcbase + t])
                    plsc.store_scatter(adjf_v, [dst_base + t], vals)
                return carry
            lax.fori_loop(0, PQ // 16, ext_body, 0)

        for half in range(2):
            def _idx(j):
                return adjf_v.at[pl.ds(half * HALF * DEG + j * CH * DEG,
                                       CH * DEG)]

            def start(j, fb, eb, sf, se):
                pltpu.async_copy(feats_hbm.at[_idx(j)], fb, sf)
                pltpu.async_copy(et_hbm.at[_idx(j)], eb, se)

            def wait(j, fb, eb, sf, se):
                pltpu.make_async_copy(feats_hbm.at[_idx(j)], fb, sf).wait()
                pltpu.make_async_copy(et_hbm.at[_idx(j)], eb, se).wait()

            def process(j, fb, eb):
                l0 = j * CH
                for s in range(CH):
                    r0 = s * DEG
                    orow = l0 + s
                    for k in range(D // L):
                        sl = pl.ds(k * L, L)
                        accA = fb[r0 + 0, sl]
                        for t in range(1, OFF):
                            accA = accA + fb[r0 + t, sl]
                        accB = fb[r0 + OFF, sl]
                        for t in range(OFF + 1, S):
                            accB = accB + fb[r0 + t, sl]
                        accC = fb[r0 + S, sl]
                        for t in range(S + 1, DEG):
                            accC = accC + fb[r0 + t, sl]
                        m0f_v[orow, sl] = accA + accB
                        m1f_v[orow, sl] = accB + accC
                    esl = pl.ds(0, L)
                    eA = eb[r0 + 0, esl]
                    for t in range(1, OFF):
                        eA = eA + eb[r0 + t, esl]
                    eB = eb[r0 + OFF, esl]
                    for t in range(OFF + 1, S):
                        eB = eB + eb[r0 + t, esl]
                    eC = eb[r0 + S, esl]
                    for t in range(S + 1, DEG):
                        eC = eC + eb[r0 + t, esl]
                    m0e_v[pl.ds(orow * D_E, D_E)] = eA + eB
                    m1e_v[pl.ds(orow * D_E, D_E)] = eB + eC

            start(0, fbufA, ebufA, semAf, semAe)
            start(1, fbufB, ebufB, semBf, semBe)

            def pair_body(jj, carry):
                j0 = jj * 2
                wait(j0, fbufA, ebufA, semAf, semAe)
                process(j0, fbufA, ebufA)

                @pl.when(jj < NCH // 2 - 1)
                def _():
                    start(j0 + 2, fbufA, ebufA, semAf, semAe)

                wait(j0 + 1, fbufB, ebufB, semBf, semBe)
                process(j0 + 1, fbufB, ebufB)

                @pl.when(jj < NCH // 2 - 1)
                def _():
                    start(j0 + 3, fbufB, ebufB, semBf, semBe)
                return carry
            lax.fori_loop(0, NCH // 2, pair_body, 0)

            row0 = base + half * HALF
            pltpu.sync_copy(m0f_v, m0f_out.at[pl.ds(row0, HALF)])
            pltpu.sync_copy(m1f_v, m1f_out.at[pl.ds(row0, HALF)])
            pltpu.sync_copy(m0e_v, m0e_out.at[pl.ds(row0 * D_E, HALF * D_E)])
            pltpu.sync_copy(m1e_v, m1e_out.at[pl.ds(row0 * D_E, HALF * D_E)])

    return sc_kernel(ids, feats, adjs2, et_exp)


def _dense_tail(f0, m0f, m1f, m0e, m1e,
                W_self_0, W_neigh_0, W_self_1, W_neigh_1, fc_w, fc_b2):
    BT = 1024
    H = D // 2

    def body(f0_r, m0f_r, m1f_r, m0e_r, m1e_r,
             ws0_r, wn0_r, ws1_r, wn1_r, fcw_r, fcb_r, out_r):
        inv_s = jnp.float32(1.0 / S)
        a0 = jnp.maximum(
            jnp.dot(f0_r[...], ws0_r[...], preferred_element_type=jnp.float32), 0.0)
        a1 = jnp.maximum(
            jnp.dot(f0_r[...], ws1_r[...], preferred_element_type=jnp.float32), 0.0)
        b0 = jnp.maximum(
            (jnp.dot(m0f_r[...], wn0_r[0:D, :], preferred_element_type=jnp.float32)
             + jnp.dot(m0e_r[...], wn0_r[D:D + D_E, :],
                       preferred_element_type=jnp.float32)) * inv_s, 0.0)
        b1 = jnp.maximum(
            (jnp.dot(m1f_r[...], wn1_r[0:D, :], preferred_element_type=jnp.float32)
             + jnp.dot(m1e_r[...], wn1_r[D:D + D_E, :],
                       preferred_element_type=jnp.float32)) * inv_s, 0.0)
        ha = (a0 + a1) * 0.5
        hb = (b0 + b1) * 0.5
        out_r[...] = jnp.maximum(
            jnp.dot(ha, fcw_r[0:H, :], preferred_element_type=jnp.float32)
            + jnp.dot(hb, fcw_r[H:D, :], preferred_element_type=jnp.float32)
            + fcb_r[...], 0.0)

    row_spec_d = pl.BlockSpec((BT, D), lambda i: (i, 0))
    row_spec_e = pl.BlockSpec((BT, D_E), lambda i: (i, 0))
    full = lambda shape: pl.BlockSpec(shape, lambda i: (0, 0))
    return pl.pallas_call(
        body,
        grid=(B // BT,),
        in_specs=[
            row_spec_d, row_spec_d, row_spec_d, row_spec_e, row_spec_e,
            full((D, H)), full((D + D_E, H)),
            full((D, H)), full((D + D_E, H)),
            full((D, N_CLASSES)), full((1, N_CLASSES)),
        ],
        out_specs=pl.BlockSpec((BT, N_CLASSES), lambda i: (i, 0)),
        out_shape=jax.ShapeDtypeStruct((B, N_CLASSES), jnp.float32),
    )(f0, m0f, m1f, m0e, m1e,
      W_self_0, W_neigh_0, W_self_1, W_neigh_1, fc_w, fc_b2)


def kernel(ids, feats, adjs, edge_emb,
           W_self_0, W_neigh_0, W_self_1, W_neigh_1, fc_w, fc_b):
    ids = ids.astype(jnp.int32)
    adjs2 = adjs.astype(jnp.int32).reshape(N_NODES * DEG // 128, 128)
    et_exp = jnp.tile(edge_emb[:N_NODES], (1, D // D_E))
    f0, m0f, m1f, m0e, m1e = _sc_aggregate(ids, feats, adjs2, et_exp)
    return _dense_tail(f0, m0f, m1f,
                       m0e.reshape(B, D_E), m1e.reshape(B, D_E),
                       W_self_0, W_neigh_0, W_self_1, W_neigh_1,
                       fc_w, fc_b.reshape(1, N_CLASSES))
